# baseline (speedup 1.0000x reference)
import functools

import jax
import jax.numpy as jnp
from jax import lax
from jax.experimental import pallas as pl
from jax.experimental.pallas import tpu as pltpu
from jax.experimental.pallas import tpu_sc as plsc

EMB_DIM = 64
VOCAB = 1000000
NUM_CORES = 2
NUM_SUBCORES = 16
NUM_WORKERS = NUM_CORES * NUM_SUBCORES

TBLK = 128
N_FULL = VOCAB // TBLK
PER_W = N_FULL // NUM_WORKERS
EXTRA = N_FULL % NUM_WORKERS

CHUNK = 128
GROUP = 5
NGROUPS = 10


def _transpose_block(slab, tbuf, ncols, rows_g, colb):
    ngrp = ncols // 16

    @pl.loop(0, EMB_DIM)
    def _(c):
        for g in range(ngrp):
            vals = slab[c, pl.ds(g * 16, 16)]
            plsc.store_scatter(tbuf, [rows_g[g], colb + c], vals)


@jax.jit
def _sc_embedding_lookup(idx3, table_t, tail2):
    mesh = plsc.VectorSubcoreMesh(core_axis_name="c", subcore_axis_name="s")

    @functools.partial(
        pl.kernel,
        mesh=mesh,
        out_type=jax.ShapeDtypeStruct((VOCAB // 2, 2 * EMB_DIM), jnp.float32),
        compiler_params=pltpu.CompilerParams(
            use_tc_tiling_on_sc=True, needs_layout_passes=False
        ),
        scratch_types=[
            pltpu.VMEM((EMB_DIM, TBLK), jnp.float32),
            pltpu.VMEM((EMB_DIM, 2 * EMB_DIM), jnp.float32),
        ],
    )
    def ktrans(tab_hbm, tail_hbm, scr_hbm, slab_v, tbuf_v):
        wid = lax.axis_index("s") * NUM_CORES + lax.axis_index("c")
        start = wid * PER_W + jnp.minimum(wid, EXTRA)
        count = PER_W + jnp.where(wid < EXTRA, 1, 0)

        iota = lax.iota(jnp.int32, 16)
        rows_g = [(g * 16 + iota) >> 1 for g in range(TBLK // 16)]
        colb = (iota & 1) * EMB_DIM

        @pl.loop(0, PER_W + 1)
        def _(k):
            @pl.when(k < count)
            def _():
                b = start + k
                pltpu.sync_copy(tab_hbm.at[:, pl.ds(b * TBLK, TBLK)], slab_v)
                _transpose_block(slab_v, tbuf_v, TBLK, rows_g, colb)
                pltpu.sync_copy(tbuf_v, scr_hbm.at[pl.ds(b * 64, 64)])

        @pl.when(wid == NUM_WORKERS - 1)
        def _():
            pltpu.sync_copy(tail_hbm, tbuf_v.at[pl.ds(0, 32)])
            pltpu.sync_copy(
                tbuf_v.at[pl.ds(0, 32)],
                scr_hbm.at[pl.ds(N_FULL * 64, 32)],
            )

    scratch_pairs = ktrans(table_t, tail2)
    scratch = scratch_pairs.reshape(VOCAB, EMB_DIM)

    n_ch = GROUP * NGROUPS
    b_per_w = n_ch * CHUNK
    B = NUM_WORKERS * b_per_w
    grp_rows = GROUP * CHUNK

    @functools.partial(
        pl.kernel,
        mesh=mesh,
        out_type=jax.ShapeDtypeStruct((B, EMB_DIM), jnp.float32),
        compiler_params=pltpu.CompilerParams(use_tc_tiling_on_sc=False),
        scratch_types=[
            pltpu.VMEM((n_ch, CHUNK), jnp.int32),
            pltpu.VMEM((grp_rows, EMB_DIM), jnp.float32),
            pltpu.VMEM((grp_rows, EMB_DIM), jnp.float32),
            pltpu.SemaphoreType.DMA,
            pltpu.SemaphoreType.DMA,
        ],
    )
    def kgather(table_hbm, idx_hbm, out_hbm, idx_v, buf_a, buf_b, sem_a, sem_b):
        wid = lax.axis_index("s") * NUM_CORES + lax.axis_index("c")
        base = wid * b_per_w
        pltpu.sync_copy(idx_hbm.at[wid], idx_v)

        def fire(g, buf, sem):
            for j in range(GROUP):
                pltpu.make_async_copy(
                    table_hbm.at[idx_v.at[g * GROUP + j]],
                    buf.at[pl.ds(j * CHUNK, CHUNK)],
                    sem,
                ).start()

        def drain(g, buf, sem):
            for j in range(GROUP):
                pltpu.make_async_copy(
                    table_hbm.at[idx_v.at[g * GROUP + j]],
                    buf.at[pl.ds(j * CHUNK, CHUNK)],
                    sem,
                ).wait()
            pltpu.sync_copy(buf, out_hbm.at[pl.ds(base + g * grp_rows, grp_rows)])

        fire(0, buf_a, sem_a)

        @pl.loop(0, NGROUPS, step=2)
        def _(g):
            @pl.when(g + 1 < NGROUPS)
            def _():
                fire(g + 1, buf_b, sem_b)

            drain(g, buf_a, sem_a)

            @pl.when(g + 2 < NGROUPS)
            def _():
                fire(g + 2, buf_a, sem_a)

            @pl.when(g + 1 < NGROUPS)
            def _():
                drain(g + 1, buf_b, sem_b)

    return kgather(scratch, idx3)


def kernel(x, table):
    B = x.shape[0] * x.shape[1]
    n_ch = GROUP * NGROUPS
    idx3 = x.reshape(NUM_WORKERS, n_ch, CHUNK)
    tail2 = table[N_FULL * TBLK :, :].reshape(32, 2 * EMB_DIM)
    out = _sc_embedding_lookup(idx3, table.T, tail2)
    return out.reshape(x.shape[0], x.shape[1], EMB_DIM)


# speedup vs baseline: 3.9504x; 3.9504x over previous
import functools

import jax
import jax.numpy as jnp
from jax import lax
from jax.experimental import pallas as pl
from jax.experimental.pallas import tpu as pltpu
from jax.experimental.pallas import tpu_sc as plsc

EMB_DIM = 64
VOCAB = 1000000
NUM_CORES = 2
NUM_SUBCORES = 16
NUM_WORKERS = NUM_CORES * NUM_SUBCORES

TSUB = 512
NSUB = 16
TW = TSUB * NSUB
TSTEPS = -(-VOCAB // TW)

CHUNK = 128
GROUP = 5
NGROUPS = 10


def _transpose_body(x_ref, o_ref, t_ref):
    for j in range(NSUB):
        blk = x_ref[:, pl.ds(j * TSUB, TSUB)]
        t_ref[...] = blk.T
        ev = t_ref[pl.Slice(0, TSUB // 2, 2), :]
        od = t_ref[pl.Slice(1, TSUB // 2, 2), :]
        o_ref[pl.ds(j * TSUB // 2, TSUB // 2), :] = jnp.concatenate(
            [ev, od], axis=1
        )


@jax.jit
def _lookup(idx3, table_t):
    scratch = pl.pallas_call(
        _transpose_body,
        grid=(TSTEPS,),
        in_specs=[pl.BlockSpec((EMB_DIM, TW), lambda i: (0, i))],
        out_specs=pl.BlockSpec((TW // 2, 2 * EMB_DIM), lambda i: (i, 0)),
        out_shape=jax.ShapeDtypeStruct(
            (TSTEPS * TW // 2, 2 * EMB_DIM), jnp.float32
        ),
        scratch_shapes=[pltpu.VMEM((TSUB, EMB_DIM), jnp.float32)],
    )(table_t)
    rows = scratch.reshape(TSTEPS * TW, EMB_DIM)

    n_ch = GROUP * NGROUPS
    b_per_w = n_ch * CHUNK
    B = NUM_WORKERS * b_per_w
    grp_rows = GROUP * CHUNK
    mesh = plsc.VectorSubcoreMesh(core_axis_name="c", subcore_axis_name="s")

    @functools.partial(
        pl.kernel,
        mesh=mesh,
        out_type=jax.ShapeDtypeStruct((B, EMB_DIM), jnp.float32),
        compiler_params=pltpu.CompilerParams(use_tc_tiling_on_sc=False),
        scratch_types=[
            pltpu.VMEM((n_ch, CHUNK), jnp.int32),
            pltpu.VMEM((grp_rows, EMB_DIM), jnp.float32),
            pltpu.VMEM((grp_rows, EMB_DIM), jnp.float32),
            pltpu.SemaphoreType.DMA,
            pltpu.SemaphoreType.DMA,
        ],
    )
    def kgather(table_hbm, idx_hbm, out_hbm, idx_v, buf_a, buf_b, sem_a, sem_b):
        wid = lax.axis_index("s") * NUM_CORES + lax.axis_index("c")
        base = wid * b_per_w
        pltpu.sync_copy(idx_hbm.at[wid], idx_v)

        def fire(g, buf, sem):
            for j in range(GROUP):
                pltpu.make_async_copy(
                    table_hbm.at[idx_v.at[g * GROUP + j]],
                    buf.at[pl.ds(j * CHUNK, CHUNK)],
                    sem,
                ).start()

        def drain(g, buf, sem):
            for j in range(GROUP):
                pltpu.make_async_copy(
                    table_hbm.at[idx_v.at[g * GROUP + j]],
                    buf.at[pl.ds(j * CHUNK, CHUNK)],
                    sem,
                ).wait()
            pltpu.sync_copy(buf, out_hbm.at[pl.ds(base + g * grp_rows, grp_rows)])

        fire(0, buf_a, sem_a)

        @pl.loop(0, NGROUPS, step=2)
        def _(g):
            @pl.when(g + 1 < NGROUPS)
            def _():
                fire(g + 1, buf_b, sem_b)

            drain(g, buf_a, sem_a)

            @pl.when(g + 2 < NGROUPS)
            def _():
                fire(g + 2, buf_a, sem_a)

            @pl.when(g + 1 < NGROUPS)
            def _():
                drain(g + 1, buf_b, sem_b)

    return kgather(rows, idx3)


def kernel(x, table):
    B = x.shape[0] * x.shape[1]
    n_ch = GROUP * NGROUPS
    idx3 = x.reshape(NUM_WORKERS, n_ch, CHUNK)
    out = _lookup(idx3, table.T)
    return out.reshape(x.shape[0], x.shape[1], EMB_DIM)
